# Initial kernel scaffold; baseline (speedup 1.0000x reference)
import jax
import jax.numpy as jnp
from jax.experimental import pallas as pl


def kernel(x, edge_attr, params, edge_index):
    raise NotImplementedError("write your pallas kernel here")


# speedup vs baseline: 3.1964x; 3.1964x over previous
import functools

import jax
import jax.numpy as jnp
from jax import lax
from jax.experimental import pallas as pl
from jax.experimental.pallas import tpu as pltpu
from jax.experimental.pallas import tpu_sc as plsc

_EN = 320000
_NN = 10000
_NPAD = 10240
_NC, _NS = 2, 16
_TE = 4000
_TN = 2000
_CHUNK = 128
_NW = _NC * _NS
_NCH = _EN // _CHUNK
_CPW = _NCH // _NW
_REM = _NCH % _NW
_RPT = _NPAD // _NS


def _ln(v, g, b):
    mu = jnp.mean(v, axis=-1, keepdims=True)
    var = jnp.mean((v - mu) ** 2, axis=-1, keepdims=True)
    return (v - mu) * lax.rsqrt(var + 1e-5) * g + b


def _leaky(v):
    return jnp.where(v >= 0, v, v * 0.01)


def _mlp(v, ws):
    n = len(ws)
    for j, (w, b) in enumerate(ws):
        v = jnp.dot(v, w, preferred_element_type=jnp.float32) + b
        if j < n - 1:
            v = _leaky(v)
    return v


def _edge_body(ea_ref, *refs):
    out_ref = refs[-1]
    w = [r[...] for r in refs[:-1]]
    enc = [(w[0], w[1]), (w[2], w[3]), (w[4], w[5])]
    g, b = w[6], w[7]
    m0 = [(w[8], w[9]), (w[10], w[11]), (w[12], w[13])]
    m1 = [(w[14], w[15]), (w[16], w[17]), (w[18], w[19])]
    ea = _ln(_mlp(ea_ref[...], enc), g, b)
    out_ref[...] = jnp.concatenate([_mlp(ea, m0), _mlp(ea, m1)], axis=-1)


def _edge_kernel(edge_attr, weights):
    full = [
        pl.BlockSpec(a.shape, lambda i, nd=a.ndim: (0,) * nd) for a in weights
    ]
    return pl.pallas_call(
        _edge_body,
        grid=(_EN // _TE,),
        in_specs=[pl.BlockSpec((_TE, 16), lambda i: (i, 0))] + full,
        out_specs=pl.BlockSpec((_TE, 128), lambda i: (i, 0)),
        out_shape=jax.ShapeDtypeStruct((_EN, 128), jnp.float32),
        compiler_params=pltpu.CompilerParams(
            dimension_semantics=("arbitrary",)
        ),
    )(edge_attr, *weights)


def _scatter_body(msg_hbm, dst_hbm, zeros_hbm, out_hbm, idx_v, rows_v, acc_sh):
    c = lax.axis_index("c")
    s = lax.axis_index("s")
    wid = c * _NS + s
    r0 = s * _RPT
    pltpu.sync_copy(
        zeros_hbm.at[pl.ds(r0, _RPT)], acc_sh.at[pl.ds(r0, _RPT)]
    )
    plsc.subcore_barrier()

    def body(j, carry):
        base = (wid + j * _NW) * _CHUNK
        pltpu.sync_copy(dst_hbm.at[pl.ds(base, _CHUNK)], idx_v)
        pltpu.sync_copy(msg_hbm.at[pl.ds(base, _CHUNK)], rows_v)
        pltpu.sync_copy(rows_v, acc_sh.at[idx_v], add=True)
        return carry

    nch = _CPW + jnp.where(wid < _REM, 1, 0)
    lax.fori_loop(0, nch, body, 0)
    plsc.subcore_barrier()
    pltpu.sync_copy(
        acc_sh.at[pl.ds(r0, _RPT)], out_hbm.at[c, pl.ds(r0, _RPT)]
    )


@functools.cache
def _build_scatter_kernel():
    return functools.partial(
        pl.kernel,
        out_type=jax.ShapeDtypeStruct((_NC, _NPAD, 128), jnp.float32),
        mesh=plsc.VectorSubcoreMesh(
            core_axis_name="c", subcore_axis_name="s", num_cores=_NC
        ),
        scratch_types=[
            pltpu.VMEM((_CHUNK,), jnp.int32),
            pltpu.VMEM((_CHUNK, 128), jnp.float32),
            pltpu.VMEM_SHARED((_NPAD, 128), jnp.float32),
        ],
    )(_scatter_body)


def _scatter_kernel(msg, dst, zeros):
    return _build_scatter_kernel()(msg, dst, zeros)


def _node_body(x_ref, p_ref, *refs):
    out_ref = refs[-1]
    w = [r[...] for r in refs[:-1]]
    enc = [(w[0], w[1]), (w[2], w[3]), (w[4], w[5])]
    eg, eb = w[6], w[7]
    dec = [(w[8], w[9]), (w[10], w[11]), (w[12], w[13])]
    y = _ln(_mlp(x_ref[...], enc), eg, eb)
    p = p_ref[0] + p_ref[1]
    aggrs = [
        lax.slice_in_dim(p, 0, 64, axis=1),
        lax.slice_in_dim(p, 64, 128, axis=1),
    ]
    k = 14
    for i in range(2):
        ng, nb = w[k], w[k + 1]
        upd = [(w[k + 2], w[k + 3]), (w[k + 4], w[k + 5]), (w[k + 6], w[k + 7])]
        og, ob = w[k + 8], w[k + 9]
        k += 10
        h = _ln(jnp.concatenate([y, aggrs[i]], axis=-1), ng, nb)
        y = y + _ln(_mlp(h, upd), og, ob)
    out_ref[...] = _mlp(y, dec)


def _node_kernel(x, partials, weights):
    full = [
        pl.BlockSpec(a.shape, lambda i, nd=a.ndim: (0,) * nd) for a in weights
    ]
    return pl.pallas_call(
        _node_body,
        grid=(_NN // _TN,),
        in_specs=[
            pl.BlockSpec((_TN, 128), lambda i: (i, 0)),
            pl.BlockSpec((2, _TN, 128), lambda i: (0, i, 0)),
        ]
        + full,
        out_specs=pl.BlockSpec((_TN, 128), lambda i: (i, 0)),
        out_shape=jax.ShapeDtypeStruct((_NN, 128), jnp.float32),
        compiler_params=pltpu.CompilerParams(
            dimension_semantics=("arbitrary",)
        ),
    )(x, partials, *weights)


def _flat_mlp(ps):
    out = []
    for wmat, bvec in ps:
        out.append(jnp.transpose(wmat))
        out.append(jnp.reshape(bvec, (1, -1)))
    return out


def _flat_ln(p):
    g, b = p
    return [jnp.reshape(g, (1, -1)), jnp.reshape(b, (1, -1))]


def kernel(x, edge_attr, params, edge_index):
    dst = edge_index[1].astype(jnp.int32)

    edge_w = (
        _flat_mlp(params['enc_edge']['mlp'])
        + _flat_ln(params['enc_edge']['ln'])
        + _flat_mlp(params['layers'][0]['msg'])
        + _flat_mlp(params['layers'][1]['msg'])
    )
    node_w = (
        _flat_mlp(params['enc']['mlp'])
        + _flat_ln(params['enc']['ln'])
        + _flat_mlp(params['dec'])
    )
    for lp in params['layers']:
        node_w += (
            _flat_ln(lp['norm']) + _flat_mlp(lp['upd']) + _flat_ln(lp['outer_ln'])
        )

    msg = _edge_kernel(edge_attr, edge_w)
    zeros = jnp.zeros((_NPAD, 128), jnp.float32)
    partials = _scatter_kernel(msg, dst, zeros)[:, :_NN, :]
    return _node_kernel(x, partials, node_w)


# speedup vs baseline: 3.8609x; 1.2079x over previous
import functools

import jax
import jax.numpy as jnp
from jax import lax
from jax.experimental import pallas as pl
from jax.experimental.pallas import tpu as pltpu
from jax.experimental.pallas import tpu_sc as plsc

_EN = 320000
_NN = 10000
_NPAD = 10240
_NC, _NS = 2, 16
_TE = 4000
_TN = 2000
_CHUNK = 128
_NW = _NC * _NS
_NCH = _EN // _CHUNK
_CPW = _NCH // _NW
_REM = _NCH % _NW
_RPT = _NPAD // _NS


def _ln(v, g, b):
    mu = jnp.mean(v, axis=-1, keepdims=True)
    var = jnp.mean((v - mu) ** 2, axis=-1, keepdims=True)
    return (v - mu) * lax.rsqrt(var + 1e-5) * g + b


def _leaky(v):
    return jnp.where(v >= 0, v, v * 0.01)


def _mlp(v, ws):
    n = len(ws)
    for j, (w, b) in enumerate(ws):
        v = jnp.dot(v, w, preferred_element_type=jnp.float32) + b
        if j < n - 1:
            v = _leaky(v)
    return v


def _edge_body(ea_ref, *refs):
    out_ref = refs[-1]
    w = [r[...] for r in refs[:-1]]
    enc = [(w[0], w[1]), (w[2], w[3]), (w[4], w[5])]
    g, b = w[6], w[7]
    m0 = [(w[8], w[9]), (w[10], w[11]), (w[12], w[13])]
    m1 = [(w[14], w[15]), (w[16], w[17]), (w[18], w[19])]
    ea = _ln(_mlp(ea_ref[...], enc), g, b)
    out_ref[...] = jnp.concatenate([_mlp(ea, m0), _mlp(ea, m1)], axis=-1)


def _edge_kernel(edge_attr, weights):
    full = [
        pl.BlockSpec(a.shape, lambda i, nd=a.ndim: (0,) * nd) for a in weights
    ]
    return pl.pallas_call(
        _edge_body,
        grid=(_EN // _TE,),
        in_specs=[pl.BlockSpec((_TE, 16), lambda i: (i, 0))] + full,
        out_specs=pl.BlockSpec((_TE, 128), lambda i: (i, 0)),
        out_shape=jax.ShapeDtypeStruct((_EN, 128), jnp.float32),
        compiler_params=pltpu.CompilerParams(
            dimension_semantics=("arbitrary",)
        ),
    )(edge_attr, *weights)


def _scatter_body(
    msg_hbm, dst_hbm, zeros_hbm, out_hbm, idx_v, rows_v, sem0, sem1, acc_sh
):
    c = lax.axis_index("c")
    s = lax.axis_index("s")
    wid = c * _NS + s
    r0 = s * _RPT
    sems = (sem0, sem1)

    def start(j, b):
        base = (wid + j * _NW) * _CHUNK
        pltpu.async_copy(dst_hbm.at[pl.ds(base, _CHUNK)], idx_v.at[b], sems[b])
        pltpu.async_copy(msg_hbm.at[pl.ds(base, _CHUNK)], rows_v.at[b], sems[b])

    def wait(b):
        pltpu.make_async_copy(
            dst_hbm.at[pl.ds(0, _CHUNK)], idx_v.at[b], sems[b]
        ).wait()
        pltpu.make_async_copy(
            msg_hbm.at[pl.ds(0, _CHUNK)], rows_v.at[b], sems[b]
        ).wait()

    for b in range(2):
        start(b, b)

    pltpu.sync_copy(
        zeros_hbm.at[pl.ds(r0, _RPT)], acc_sh.at[pl.ds(r0, _RPT)]
    )
    plsc.subcore_barrier()

    def body(it, carry):
        j0 = it * 2
        for b in range(2):
            j = j0 + b
            wait(b)
            pltpu.sync_copy(rows_v.at[b], acc_sh.at[idx_v.at[b]], add=True)

            @pl.when(j + 2 < _CPW)
            def _():
                start(j + 2, b)

        return carry

    lax.fori_loop(0, _CPW // 2, body, 0)

    @pl.when(wid < _REM)
    def _():
        base = (wid + _CPW * _NW) * _CHUNK
        pltpu.sync_copy(dst_hbm.at[pl.ds(base, _CHUNK)], idx_v.at[0])
        pltpu.sync_copy(msg_hbm.at[pl.ds(base, _CHUNK)], rows_v.at[0])
        pltpu.sync_copy(rows_v.at[0], acc_sh.at[idx_v.at[0]], add=True)

    plsc.subcore_barrier()
    pltpu.sync_copy(
        acc_sh.at[pl.ds(r0, _RPT)], out_hbm.at[c, pl.ds(r0, _RPT)]
    )


@functools.cache
def _build_scatter_kernel():
    return functools.partial(
        pl.kernel,
        out_type=jax.ShapeDtypeStruct((_NC, _NPAD, 128), jnp.float32),
        mesh=plsc.VectorSubcoreMesh(
            core_axis_name="c", subcore_axis_name="s", num_cores=_NC
        ),
        scratch_types=[
            pltpu.VMEM((2, _CHUNK), jnp.int32),
            pltpu.VMEM((2, _CHUNK, 128), jnp.float32),
            pltpu.SemaphoreType.DMA,
            pltpu.SemaphoreType.DMA,
            pltpu.VMEM_SHARED((_NPAD, 128), jnp.float32),
        ],
    )(_scatter_body)


def _scatter_kernel(msg, dst, zeros):
    return _build_scatter_kernel()(msg, dst, zeros)


def _node_body(x_ref, p_ref, *refs):
    out_ref = refs[-1]
    w = [r[...] for r in refs[:-1]]
    enc = [(w[0], w[1]), (w[2], w[3]), (w[4], w[5])]
    eg, eb = w[6], w[7]
    dec = [(w[8], w[9]), (w[10], w[11]), (w[12], w[13])]
    y = _ln(_mlp(x_ref[...], enc), eg, eb)
    p = p_ref[0] + p_ref[1]
    aggrs = [
        lax.slice_in_dim(p, 0, 64, axis=1),
        lax.slice_in_dim(p, 64, 128, axis=1),
    ]
    k = 14
    for i in range(2):
        ng, nb = w[k], w[k + 1]
        upd = [(w[k + 2], w[k + 3]), (w[k + 4], w[k + 5]), (w[k + 6], w[k + 7])]
        og, ob = w[k + 8], w[k + 9]
        k += 10
        h = _ln(jnp.concatenate([y, aggrs[i]], axis=-1), ng, nb)
        y = y + _ln(_mlp(h, upd), og, ob)
    out_ref[...] = _mlp(y, dec)


def _node_kernel(x, partials, weights):
    full = [
        pl.BlockSpec(a.shape, lambda i, nd=a.ndim: (0,) * nd) for a in weights
    ]
    return pl.pallas_call(
        _node_body,
        grid=(_NN // _TN,),
        in_specs=[
            pl.BlockSpec((_TN, 128), lambda i: (i, 0)),
            pl.BlockSpec((_NC, _TN, 128), lambda i: (0, i, 0)),
        ]
        + full,
        out_specs=pl.BlockSpec((_TN, 128), lambda i: (i, 0)),
        out_shape=jax.ShapeDtypeStruct((_NN, 128), jnp.float32),
        compiler_params=pltpu.CompilerParams(
            dimension_semantics=("arbitrary",)
        ),
    )(x, partials, *weights)


def _flat_mlp(ps):
    out = []
    for wmat, bvec in ps:
        out.append(jnp.transpose(wmat))
        out.append(jnp.reshape(bvec, (1, -1)))
    return out


def _flat_ln(p):
    g, b = p
    return [jnp.reshape(g, (1, -1)), jnp.reshape(b, (1, -1))]


def kernel(x, edge_attr, params, edge_index):
    dst = edge_index[1].astype(jnp.int32)

    edge_w = (
        _flat_mlp(params['enc_edge']['mlp'])
        + _flat_ln(params['enc_edge']['ln'])
        + _flat_mlp(params['layers'][0]['msg'])
        + _flat_mlp(params['layers'][1]['msg'])
    )
    node_w = (
        _flat_mlp(params['enc']['mlp'])
        + _flat_ln(params['enc']['ln'])
        + _flat_mlp(params['dec'])
    )
    for lp in params['layers']:
        node_w += (
            _flat_ln(lp['norm']) + _flat_mlp(lp['upd']) + _flat_ln(lp['outer_ln'])
        )

    msg = _edge_kernel(edge_attr, edge_w)
    zeros = jnp.zeros((_NPAD, 128), jnp.float32)
    partials = _scatter_kernel(msg, dst, zeros)
    return _node_kernel(x, partials, node_w)
